# baseline (speedup 1.0000x reference)
import functools

import jax
import jax.numpy as jnp
from jax import lax
from jax.experimental import pallas as pl
from jax.experimental.pallas import tpu as pltpu
from jax.experimental.pallas import tpu_sc as plsc

_D = 64
_PAIR = 2 * _D
_SCALE = 64 ** 0.5
_LANES = 16
_BB = 128
_P = 128

_info = plsc.get_sparse_core_info()
_NC, _NS = _info.num_cores, _info.num_subcores
_NW = _NC * _NS


def _make_stage(n_rows: int):
    n_pairs = n_rows // 2
    n_blocks = n_pairs // _P
    n_tail = n_pairs - n_blocks * _P
    per_w_max = -(-n_blocks // _NW)
    mesh = plsc.VectorSubcoreMesh(core_axis_name="c", subcore_axis_name="s")

    @functools.partial(
        pl.kernel,
        mesh=mesh,
        out_type=jax.ShapeDtypeStruct((n_pairs, _PAIR), jnp.float32),
        scratch_types=[
            pltpu.VMEM((_D, 2 * _P), jnp.float32),
            pltpu.VMEM((_D, 2 * _P), jnp.float32),
            pltpu.VMEM((_P, _PAIR), jnp.float32),
            pltpu.VMEM((_P, _PAIR), jnp.float32),
            pltpu.VMEM((_D, _D), jnp.float32),
            pltpu.VMEM((_P // 4, _PAIR), jnp.float32),
            pltpu.SemaphoreType.DMA,
            pltpu.SemaphoreType.DMA,
            pltpu.SemaphoreType.DMA,
            pltpu.SemaphoreType.DMA,
        ],
        compiler_params=pltpu.CompilerParams(needs_layout_passes=False),
    )
    def stage_kernel(tt_hbm, out_hbm, in0, in1, o0, o1, tin, tout,
                     gi0, gi1, go0, go1):
        ibuf, obuf = (in0, in1), (o0, o1)
        isem, osem = (gi0, gi1), (go0, go1)
        wid = lax.axis_index("s") * _NC + lax.axis_index("c")
        iota = lax.iota(jnp.int32, _LANES)

        def blk_id(k):
            return k * _NW + wid

        def in_start(k, b):
            col0 = pl.multiple_of(blk_id(k) * 2 * _P, 2 * _P)
            pltpu.async_copy(
                tt_hbm.at[:, pl.ds(col0, 2 * _P)], ibuf[b], isem[b])

        def in_wait(b):
            pltpu.make_async_copy(
                tt_hbm.at[:, pl.ds(0, 2 * _P)], ibuf[b], isem[b]).wait()

        def out_start(k, b):
            pltpu.async_copy(
                obuf[b], out_hbm.at[pl.ds(blk_id(k) * _P, _P)], osem[b])

        def out_wait(b):
            pltpu.make_async_copy(
                obuf[b], out_hbm.at[pl.ds(0, _P)], osem[b]).wait()

        def transpose(b):
            src, dst = ibuf[b], obuf[b]

            @plsc.parallel_loop(0, _P, 1)
            def _(q):
                for h in range(2):
                    col = jnp.full((_LANES,), 2 * q + h, jnp.int32)
                    for m in range(_D // _LANES):
                        v = plsc.load_gather(src, [iota + m * _LANES, col])
                        dst[q, pl.ds(h * _D + m * _LANES, _LANES)] = v

        @pl.when(wid < n_blocks)
        def _():
            in_start(0, 0)

        @pl.when(_NW + wid < n_blocks)
        def _():
            in_start(1, 1)

        def body(i, carry):
            for b in range(2):
                k = 2 * i + b

                @pl.when(blk_id(k) < n_blocks)
                def _():
                    in_wait(b)

                    @pl.when(k >= 2)
                    def _():
                        out_wait(b)

                    transpose(b)
                    out_start(k, b)

                    @pl.when(blk_id(k + 2) < n_blocks)
                    def _():
                        in_start(k + 2, b)
            return carry

        lax.fori_loop(0, (per_w_max + 1) // 2, body, 0)

        @pl.when(wid < n_blocks)
        def _():
            out_wait(0)

        @pl.when(_NW + wid < n_blocks)
        def _():
            out_wait(1)

        if n_tail:
            assert (n_blocks * 2 * _P) % 128 == 0 and 2 * n_tail == _D
            @pl.when(wid == _NW - 1)
            def _():
                pltpu.sync_copy(
                    tt_hbm.at[:, pl.ds(n_blocks * 2 * _P, 2 * n_tail)], tin)

                def tail_q(q, carry):
                    for h in range(2):
                        col = jnp.full((_LANES,), 2 * q + h, jnp.int32)
                        for m in range(_D // _LANES):
                            v = plsc.load_gather(
                                tin, [iota + m * _LANES, col])
                            tout[q, pl.ds(h * _D + m * _LANES, _LANES)] = v
                    return carry

                lax.fori_loop(0, n_tail, tail_q, 0)
                pltpu.sync_copy(
                    tout, out_hbm.at[pl.ds(n_blocks * _P, n_tail)])

    return stage_kernel


def _make_gather(n_batch: int, n_time: int):
    assert n_batch == _BB * _NW
    per_w = _BB * n_time
    n_groups = _BB // _LANES
    nbuf = 4
    mesh = plsc.VectorSubcoreMesh(core_axis_name="c", subcore_axis_name="s")

    @functools.partial(
        pl.kernel,
        mesh=mesh,
        out_type=jax.ShapeDtypeStruct(
            (n_time, _D // 8, _NW, 8, _BB), jnp.float32),
        scratch_types=(
            [pltpu.VMEM((per_w,), jnp.int32)]
            + [pltpu.VMEM((_BB,), jnp.int32) for _ in range(2 * nbuf)]
            + [pltpu.VMEM((_BB, _PAIR), jnp.float32) for _ in range(nbuf)]
            + [pltpu.VMEM((_D, _BB), jnp.float32) for _ in range(nbuf)]
            + [pltpu.SemaphoreType.DMA for _ in range(2 * nbuf)]
        ),
        compiler_params=pltpu.CompilerParams(needs_layout_passes=False),
    )
    def gather_kernel(idx_hbm, table_hbm, out_hbm, span, *bufs):
        pbuf = bufs[0:nbuf]
        hbuf = bufs[nbuf:2 * nbuf]
        rbuf = bufs[2 * nbuf:3 * nbuf]
        kbuf = bufs[3 * nbuf:4 * nbuf]
        gsem = bufs[4 * nbuf:5 * nbuf]
        ssem = bufs[5 * nbuf:6 * nbuf]
        wid = lax.axis_index("s") * _NC + lax.axis_index("c")
        iota = lax.iota(jnp.int32, _LANES)

        pltpu.sync_copy(idx_hbm.at[pl.ds(wid * per_w, per_w)], span)

        def prep(t, b):
            for m in range(n_groups):
                pos = (iota + m * _LANES) * n_time + t
                vals = plsc.load_gather(span, [pos])
                sl = pl.ds(m * _LANES, _LANES)
                pbuf[b][sl] = lax.shift_right_logical(vals, 1)
                hbuf[b][sl] = lax.bitwise_and(vals, 1) * _D

        def gather_start(b):
            pltpu.async_copy(table_hbm.at[pbuf[b]], rbuf[b], gsem[b])

        def gather_wait(b):
            pltpu.make_async_copy(
                table_hbm.at[pbuf[b]], rbuf[b], gsem[b]).wait()

        def transpose_block(b):
            rows = rbuf[b]
            blk = kbuf[b]
            hv = [hbuf[b][pl.ds(m * _LANES, _LANES)] for m in range(n_groups)]
            bv = [iota + m * _LANES for m in range(n_groups)]

            @plsc.parallel_loop(0, _D, 1, unroll=2)
            def _(c):
                for m in range(n_groups):
                    v = plsc.load_gather(rows, [bv[m], hv[m] + c])
                    blk[c, pl.ds(m * _LANES, _LANES)] = v * _SCALE

        def scatter_start(t, b):
            for cb in range(_D // 8):
                pltpu.async_copy(
                    kbuf[b].at[pl.ds(cb * 8, 8)],
                    out_hbm.at[t, cb, wid], ssem[b])

        def scatter_wait(b):
            for _ in range(_D // 8):
                pltpu.make_async_copy(
                    kbuf[b].at[pl.ds(0, 8)], out_hbm.at[0, 0, 0],
                    ssem[b]).wait()

        for b in range(nbuf):
            prep(b, b)
            gather_start(b)

        def body(i, carry):
            for b in range(nbuf):
                t = nbuf * i + b
                gather_wait(b)

                @pl.when(t >= nbuf)
                def _():
                    scatter_wait(b)

                transpose_block(b)
                scatter_start(t, b)

                @pl.when(t + nbuf < n_time)
                def _():
                    prep(t + nbuf, b)
                    gather_start(b)
            return carry

        lax.fori_loop(0, n_time // nbuf, body, 0)
        for b in range(nbuf):
            scatter_wait(b)

    return gather_kernel


def kernel(inp_tokens, emb_table):
    n_batch, n_time = inp_tokens.shape
    idx = inp_tokens.reshape(-1).astype(jnp.int32)
    table2 = _make_stage(emb_table.shape[0])(emb_table.T)
    out_p = _make_gather(n_batch, n_time)(idx, table2)
    return out_p.transpose(2, 4, 0, 1, 3).reshape(n_batch, n_time, _D)


# speedup vs baseline: 1.7107x; 1.7107x over previous
import functools

import jax
import jax.numpy as jnp
from jax import lax
from jax.experimental import pallas as pl
from jax.experimental.pallas import tpu as pltpu
from jax.experimental.pallas import tpu_sc as plsc

_D = 64
_W = 2 * _D
_SCALE = 64 ** 0.5
_LANES = 16

_info = plsc.get_sparse_core_info()
_NC, _NS = _info.num_cores, _info.num_subcores
_NW = _NC * _NS


def _make_gather(n_batch: int, n_time: int, chunk_rows: int):
    n_idx = n_batch * n_time
    chunk = chunk_rows * n_time
    assert n_idx % (_NW * chunk) == 0
    per_w = n_idx // _NW
    rows_w = per_w // n_time
    n_chunks = per_w // chunk
    assert n_chunks % 2 == 0 and n_chunks >= 4
    mesh = plsc.VectorSubcoreMesh(core_axis_name="c", subcore_axis_name="s")

    @functools.partial(
        pl.kernel,
        mesh=mesh,
        out_type=jax.ShapeDtypeStruct((n_batch, n_time, _W), jnp.float32),
        scratch_types=[
            pltpu.VMEM((chunk,), jnp.int32),
            pltpu.VMEM((chunk,), jnp.int32),
            pltpu.VMEM((chunk,), jnp.int32),
            pltpu.VMEM((chunk,), jnp.int32),
            pltpu.VMEM((chunk, _D), jnp.float32),
            pltpu.VMEM((chunk, _D), jnp.float32),
            pltpu.SemaphoreType.DMA,
            pltpu.SemaphoreType.DMA,
            pltpu.SemaphoreType.DMA,
            pltpu.SemaphoreType.DMA,
            pltpu.SemaphoreType.DMA,
            pltpu.SemaphoreType.DMA,
        ],
        compiler_params=pltpu.CompilerParams(use_tc_tiling_on_sc=False),
    )
    def gather_kernel(idx_hbm, table_hbm, out_hbm,
                      i0, i1, q0, q1, r0, r1, is0, is1, gs0, gs1, ss0, ss1):
        ibuf, qbuf, rbuf = (i0, i1), (q0, q1), (r0, r1)
        isem, gsem, ssem = (is0, is1), (gs0, gs1), (ss0, ss1)
        wid = lax.axis_index("s") * _NC + lax.axis_index("c")
        base = wid * per_w

        def off(g):
            return base + g * chunk

        def idx_start(g, b):
            pltpu.async_copy(idx_hbm.at[pl.ds(off(g), chunk)], ibuf[b], isem[b])

        def idx_wait(b):
            pltpu.make_async_copy(
                idx_hbm.at[pl.ds(base, chunk)], ibuf[b], isem[b]).wait()

        def gather_start(b):
            def dbl(j, carry):
                sl = pl.ds(j * _LANES, _LANES)
                qbuf[b][sl] = ibuf[b][sl] * 2
                return carry

            lax.fori_loop(0, chunk // _LANES, dbl, 0)
            pltpu.async_copy(table_hbm.at[qbuf[b]], rbuf[b], gsem[b])

        def gather_wait(b):
            pltpu.make_async_copy(
                table_hbm.at[qbuf[b]], rbuf[b], gsem[b]).wait()

        def scatter_start(g, b):
            row0 = wid * rows_w + g * chunk_rows
            for k in range(chunk_rows):
                pltpu.async_copy(
                    rbuf[b].at[pl.ds(k * n_time, n_time)],
                    out_hbm.at[row0 + k, :, pl.ds(0, _D)], ssem[b])

        def scatter_wait(b):
            for _ in range(chunk_rows):
                pltpu.make_async_copy(
                    rbuf[b].at[pl.ds(0, n_time)],
                    out_hbm.at[0, :, pl.ds(0, _D)], ssem[b]).wait()

        def scale(b):
            rows = rbuf[b]

            @plsc.parallel_loop(0, chunk, 1, unroll=4)
            def _(r):
                for c in range(_D // _LANES):
                    sl = pl.ds(c * _LANES, _LANES)
                    rows[r, sl] = rows[r, sl] * _SCALE

        idx_start(0, 0)
        idx_start(1, 1)
        idx_wait(0)
        gather_start(0)

        def body(i, carry):
            for b in range(2):
                g = 2 * i + b
                nb = 1 - b

                @pl.when(g + 1 < n_chunks)
                def _():
                    idx_wait(nb)

                    @pl.when(g >= 1)
                    def _():
                        scatter_wait(nb)

                    gather_start(nb)

                gather_wait(b)

                @pl.when(g + 2 < n_chunks)
                def _():
                    idx_start(g + 2, b)

                scale(b)
                scatter_start(g, b)
            return carry

        lax.fori_loop(0, n_chunks // 2, body, 0)
        scatter_wait(0)
        scatter_wait(1)

    return gather_kernel


def kernel(inp_tokens, emb_table):
    n_batch, n_time = inp_tokens.shape
    idx = inp_tokens.reshape(-1).astype(jnp.int32)
    table_dup = jnp.pad(emb_table, ((0, 0), (0, _W - _D)))
    table_dup = table_dup.reshape(2 * emb_table.shape[0], _D)
    out128 = _make_gather(n_batch, n_time, 2)(idx, table_dup)
    return out128[:, :, :_D]
